# baseline (speedup 1.0000x reference)
import functools

import jax
import jax.numpy as jnp
from jax import lax
from jax.experimental import pallas as pl
from jax.experimental.pallas import tpu as pltpu
from jax.experimental.pallas import tpu_sc as plsc

HID = 2048
EPS = 1e-05

NUM_CORES = 2
NUM_SUBCORES = 16
NUM_WORKERS = NUM_CORES * NUM_SUBCORES
LANES = 16
VECS_PER_ROW = HID // LANES

CHUNK = 8
NBUF = 2
NSPLIT = 4
LN_BLK = 512


def _make_gather_sum(num_tokens):
    tok_per_w = num_tokens // NUM_WORKERS
    n_chunks = tok_per_w // CHUNK
    n_outer = n_chunks // NBUF
    mesh = plsc.VectorSubcoreMesh(
        core_axis_name="c", subcore_axis_name="s")

    @functools.partial(
        pl.kernel,
        out_type=jax.ShapeDtypeStruct((num_tokens, HID), jnp.float32),
        mesh=mesh,
        scratch_types=[
            pltpu.VMEM((tok_per_w,), jnp.int32),
            pltpu.VMEM((tok_per_w,), jnp.int32),
            pltpu.VMEM((NBUF, CHUNK, HID), jnp.float32),
            pltpu.VMEM((NBUF, CHUNK, HID), jnp.float32),
            pltpu.VMEM((NBUF, CHUNK, HID), jnp.float32),
            [pltpu.SemaphoreType.DMA] * NBUF,
            [pltpu.SemaphoreType.DMA] * NBUF,
            [pltpu.SemaphoreType.DMA] * NBUF,
        ],
    )
    def gather_sum(ids_hbm, pids_hbm, wtab_hbm, ptab_hbm, out_hbm,
                   idx_v, pidx_v, wbuf, pbuf, obuf, sem_w, sem_p, sem_o):
        wid = lax.axis_index("s") * NUM_CORES + lax.axis_index("c")
        base = wid * tok_per_w
        pltpu.sync_copy(ids_hbm.at[pl.ds(base, tok_per_w)], idx_v)
        pltpu.sync_copy(pids_hbm.at[pl.ds(base, tok_per_w)], pidx_v)

        def fire_gathers(c, b):
            off = c * CHUNK
            pltpu.async_copy(
                wtab_hbm.at[idx_v.at[pl.ds(off, CHUNK)]], wbuf.at[b],
                sem_w[b])
            pltpu.async_copy(
                ptab_hbm.at[pidx_v.at[pl.ds(off, CHUNK)]], pbuf.at[b],
                sem_p[b])

        for b in range(NBUF):
            fire_gathers(b, b)

        def outer_body(o, carry):
            for b in range(NBUF):
                c = o * NBUF + b
                pltpu.make_async_copy(
                    wtab_hbm.at[idx_v.at[pl.ds(0, CHUNK)]], wbuf.at[b],
                    sem_w[b]).wait()
                pltpu.make_async_copy(
                    ptab_hbm.at[pidx_v.at[pl.ds(0, CHUNK)]], pbuf.at[b],
                    sem_p[b]).wait()
                @pl.when(o > 0)
                def _():
                    pltpu.make_async_copy(
                        obuf.at[b], out_hbm.at[pl.ds(0, CHUNK)],
                        sem_o[b]).wait()

                def v_body(v):
                    sl = pl.ds(v * LANES, LANES)
                    for r in range(CHUNK):
                        obuf[b, r, sl] = wbuf[b, r, sl] + pbuf[b, r, sl]

                plsc.parallel_loop(0, VECS_PER_ROW, 1, unroll=4)(v_body)

                pltpu.async_copy(
                    obuf.at[b], out_hbm.at[pl.ds(base + c * CHUNK, CHUNK)],
                    sem_o[b])

                @pl.when(c + NBUF < n_chunks)
                def _():
                    fire_gathers(c + NBUF, b)
            return carry

        lax.fori_loop(0, n_outer, outer_body, 0, unroll=False)
        for b in range(NBUF):
            pltpu.make_async_copy(
                obuf.at[b], out_hbm.at[pl.ds(0, CHUNK)], sem_o[b]).wait()

    return gather_sum


def _ln_body(x_ref, t_ref, g_ref, b_ref, o_ref):
    e = x_ref[...] + t_ref[...]
    mu = jnp.mean(e, axis=-1, keepdims=True)
    d = e - mu
    var = jnp.mean(d * d, axis=-1, keepdims=True)
    o_ref[...] = d * lax.rsqrt(var + EPS) * g_ref[...] + b_ref[...]


def _ln_body_acc(x_ref, t_ref, g_ref, b_ref, buf_ref, o_ref):
    del buf_ref
    _ln_body(x_ref, t_ref, g_ref, b_ref, o_ref)


def _layernorm_slice(summed, type_row, gamma, beta, n_total, blk_off, buf):
    n = summed.shape[0]
    in_specs = [
        pl.BlockSpec((LN_BLK, HID), lambda i: (i, 0)),
        pl.BlockSpec((1, HID), lambda i: (0, 0)),
        pl.BlockSpec((1, HID), lambda i: (0, 0)),
        pl.BlockSpec((1, HID), lambda i: (0, 0)),
    ]
    args = [summed, type_row, gamma, beta]
    kwargs = {}
    body = _ln_body
    if buf is not None:
        in_specs.append(pl.BlockSpec(memory_space=pl.ANY))
        args.append(buf)
        kwargs["input_output_aliases"] = {4: 0}
        body = _ln_body_acc
    return pl.pallas_call(
        body,
        grid=(n // LN_BLK,),
        in_specs=in_specs,
        out_specs=pl.BlockSpec((LN_BLK, HID), lambda i: (i + blk_off, 0)),
        out_shape=jax.ShapeDtypeStruct((n_total, HID), jnp.float32),
        **kwargs,
    )(*args)


def kernel(input_ids, position_ids, word_table, pos_table, type_table,
           gamma, beta):
    b, s = input_ids.shape
    n = b * s
    ids = input_ids.reshape(n)
    pids = position_ids.reshape(n)
    h = n // NSPLIT
    gs = _make_gather_sum(h)
    type_row = type_table[0:1, :]
    g2 = gamma.reshape(1, HID)
    b2 = beta.reshape(1, HID)

    summed = [
        gs(ids[q * h:(q + 1) * h], pids[q * h:(q + 1) * h],
           word_table, pos_table)
        for q in range(NSPLIT)
    ]
    buf = None
    for q in range(NSPLIT):
        buf = _layernorm_slice(
            summed[q], type_row, g2, b2, n, q * (h // LN_BLK), buf)
    return buf.reshape(b, s, HID)


# speedup vs baseline: 1.0844x; 1.0844x over previous
import functools

import jax
import jax.numpy as jnp
from jax import lax
from jax.experimental import pallas as pl
from jax.experimental.pallas import tpu as pltpu
from jax.experimental.pallas import tpu_sc as plsc

HID = 2048
EPS = 1e-05

NUM_CORES = 2
NUM_SUBCORES = 16
NUM_WORKERS = NUM_CORES * NUM_SUBCORES
LANES = 16
VECS_PER_ROW = HID // LANES

CHUNK = 8
NBUF = 3
NSPLIT = 1
LN_BLK = 512


def _make_gather_sum(num_tokens):
    tok_per_w = num_tokens // NUM_WORKERS
    n_chunks = tok_per_w // CHUNK
    n_main = (n_chunks // NBUF) * NBUF
    n_outer = n_main // NBUF
    mesh = plsc.VectorSubcoreMesh(
        core_axis_name="c", subcore_axis_name="s")

    @functools.partial(
        pl.kernel,
        out_type=jax.ShapeDtypeStruct((num_tokens, HID), jnp.float32),
        mesh=mesh,
        scratch_types=[
            pltpu.VMEM((tok_per_w,), jnp.int32),
            pltpu.VMEM((tok_per_w,), jnp.int32),
            pltpu.VMEM((NBUF, CHUNK, HID), jnp.float32),
            pltpu.VMEM((NBUF, CHUNK, HID), jnp.float32),
            pltpu.VMEM((CHUNK, HID), jnp.float32),
            [pltpu.SemaphoreType.DMA] * NBUF,
            [pltpu.SemaphoreType.DMA] * NBUF,
            pltpu.SemaphoreType.DMA,
        ],
    )
    def gather_sum(ids_hbm, pids_hbm, wtab_hbm, ptab_hbm, out_hbm,
                   idx_v, pidx_v, wbuf, pbuf, obuf, sem_w, sem_p, sem_o):
        wid = lax.axis_index("s") * NUM_CORES + lax.axis_index("c")
        base = wid * tok_per_w
        pltpu.sync_copy(ids_hbm.at[pl.ds(base, tok_per_w)], idx_v)
        pltpu.sync_copy(pids_hbm.at[pl.ds(base, tok_per_w)], pidx_v)

        def fire_gathers(c, b):
            off = c * CHUNK
            pltpu.async_copy(
                wtab_hbm.at[idx_v.at[pl.ds(off, CHUNK)]], wbuf.at[b],
                sem_w[b])
            pltpu.async_copy(
                ptab_hbm.at[pidx_v.at[pl.ds(off, CHUNK)]], pbuf.at[b],
                sem_p[b])

        def wait_writeback():
            pltpu.make_async_copy(
                obuf, out_hbm.at[pl.ds(0, CHUNK)], sem_o).wait()

        def process(c, b, refill):
            pltpu.make_async_copy(
                wtab_hbm.at[idx_v.at[pl.ds(0, CHUNK)]], wbuf.at[b],
                sem_w[b]).wait()
            pltpu.make_async_copy(
                ptab_hbm.at[pidx_v.at[pl.ds(0, CHUNK)]], pbuf.at[b],
                sem_p[b]).wait()

            if isinstance(c, int):
                wait_writeback()
            else:
                @pl.when(c > 0)
                def _():
                    wait_writeback()

            def v_body(v):
                sl = pl.ds(v * LANES, LANES)
                for r in range(CHUNK):
                    obuf[r, sl] = wbuf[b, r, sl] + pbuf[b, r, sl]

            plsc.parallel_loop(0, VECS_PER_ROW, 1, unroll=4)(v_body)

            pltpu.async_copy(
                obuf, out_hbm.at[pl.ds(base + c * CHUNK, CHUNK)], sem_o)
            if refill:
                @pl.when(c + NBUF < n_chunks)
                def _():
                    fire_gathers(c + NBUF, b)

        for b in range(NBUF):
            fire_gathers(b, b)

        def outer_body(o, carry):
            for b in range(NBUF):
                process(o * NBUF + b, b, refill=True)
            return carry

        lax.fori_loop(0, n_outer, outer_body, 0, unroll=False)
        for c in range(n_main, n_chunks):
            process(c, c % NBUF, refill=False)
        wait_writeback()

    return gather_sum


def _ln_body(x_ref, t_ref, g_ref, b_ref, o_ref):
    e = x_ref[...] + t_ref[...]
    mu = jnp.mean(e, axis=-1, keepdims=True)
    d = e - mu
    var = jnp.mean(d * d, axis=-1, keepdims=True)
    o_ref[...] = d * lax.rsqrt(var + EPS) * g_ref[...] + b_ref[...]


def _ln_body_acc(x_ref, t_ref, g_ref, b_ref, buf_ref, o_ref):
    del buf_ref
    _ln_body(x_ref, t_ref, g_ref, b_ref, o_ref)


def _layernorm_slice(summed, type_row, gamma, beta, n_total, blk_off, buf):
    n = summed.shape[0]
    in_specs = [
        pl.BlockSpec((LN_BLK, HID), lambda i: (i, 0)),
        pl.BlockSpec((1, HID), lambda i: (0, 0)),
        pl.BlockSpec((1, HID), lambda i: (0, 0)),
        pl.BlockSpec((1, HID), lambda i: (0, 0)),
    ]
    args = [summed, type_row, gamma, beta]
    kwargs = {}
    body = _ln_body
    if buf is not None:
        in_specs.append(pl.BlockSpec(memory_space=pl.ANY))
        args.append(buf)
        kwargs["input_output_aliases"] = {4: 0}
        body = _ln_body_acc
    return pl.pallas_call(
        body,
        grid=(n // LN_BLK,),
        in_specs=in_specs,
        out_specs=pl.BlockSpec((LN_BLK, HID), lambda i: (i + blk_off, 0)),
        out_shape=jax.ShapeDtypeStruct((n_total, HID), jnp.float32),
        **kwargs,
    )(*args)


def kernel(input_ids, position_ids, word_table, pos_table, type_table,
           gamma, beta):
    b, s = input_ids.shape
    n = b * s
    ids = input_ids.reshape(n)
    pids = position_ids.reshape(n)
    h = n // NSPLIT
    gs = _make_gather_sum(h)
    type_row = type_table[0:1, :]
    g2 = gamma.reshape(1, HID)
    b2 = beta.reshape(1, HID)

    summed = [
        gs(ids[q * h:(q + 1) * h], pids[q * h:(q + 1) * h],
           word_table, pos_table)
        for q in range(NSPLIT)
    ]
    buf = None
    for q in range(NSPLIT):
        buf = _layernorm_slice(
            summed[q], type_row, g2, b2, n, q * (h // LN_BLK), buf)
    return buf.reshape(b, s, HID)


# speedup vs baseline: 1.1021x; 1.0163x over previous
import functools

import jax
import jax.numpy as jnp
from jax import lax
from jax.experimental import pallas as pl
from jax.experimental.pallas import tpu as pltpu
from jax.experimental.pallas import tpu_sc as plsc

HID = 2048
EPS = 1e-05

NUM_CORES = 2
NUM_SUBCORES = 16
NUM_WORKERS = NUM_CORES * NUM_SUBCORES
LANES = 16
VECS_PER_ROW = HID // LANES

CHUNK = 8
NBUF = 2
NSPLIT = 1
LN_BLK = 1024


def _make_gather_sum(num_tokens):
    tok_per_w = num_tokens // NUM_WORKERS
    n_chunks = tok_per_w // CHUNK
    n_outer = n_chunks // NBUF
    mesh = plsc.VectorSubcoreMesh(
        core_axis_name="c", subcore_axis_name="s")

    @functools.partial(
        pl.kernel,
        out_type=jax.ShapeDtypeStruct((num_tokens, HID), jnp.float32),
        mesh=mesh,
        scratch_types=[
            pltpu.VMEM((tok_per_w,), jnp.int32),
            pltpu.VMEM((tok_per_w,), jnp.int32),
            pltpu.VMEM((NBUF, CHUNK, HID), jnp.float32),
            pltpu.VMEM((NBUF, CHUNK, HID), jnp.float32),
            pltpu.VMEM((NBUF, CHUNK, HID), jnp.float32),
            [pltpu.SemaphoreType.DMA] * NBUF,
            [pltpu.SemaphoreType.DMA] * NBUF,
            [pltpu.SemaphoreType.DMA] * NBUF,
        ],
    )
    def gather_sum(ids_hbm, pids_hbm, wtab_hbm, ptab_hbm, out_hbm,
                   idx_v, pidx_v, wbuf, pbuf, obuf, sem_w, sem_p, sem_o):
        wid = lax.axis_index("s") * NUM_CORES + lax.axis_index("c")
        base = wid * tok_per_w
        pltpu.sync_copy(ids_hbm.at[pl.ds(base, tok_per_w)], idx_v)
        pltpu.sync_copy(pids_hbm.at[pl.ds(base, tok_per_w)], pidx_v)

        def fire_gathers(c, b):
            off = c * CHUNK
            pltpu.async_copy(
                wtab_hbm.at[idx_v.at[pl.ds(off, CHUNK)]], wbuf.at[b],
                sem_w[b])
            pltpu.async_copy(
                ptab_hbm.at[pidx_v.at[pl.ds(off, CHUNK)]], pbuf.at[b],
                sem_p[b])

        for b in range(NBUF):
            fire_gathers(b, b)

        def outer_body(o, carry):
            for b in range(NBUF):
                c = o * NBUF + b
                pltpu.make_async_copy(
                    wtab_hbm.at[idx_v.at[pl.ds(0, CHUNK)]], wbuf.at[b],
                    sem_w[b]).wait()
                pltpu.make_async_copy(
                    ptab_hbm.at[pidx_v.at[pl.ds(0, CHUNK)]], pbuf.at[b],
                    sem_p[b]).wait()
                @pl.when(o > 0)
                def _():
                    pltpu.make_async_copy(
                        obuf.at[b], out_hbm.at[pl.ds(0, CHUNK)],
                        sem_o[b]).wait()

                def v_body(v):
                    sl = pl.ds(v * LANES, LANES)
                    for r in range(CHUNK):
                        obuf[b, r, sl] = wbuf[b, r, sl] + pbuf[b, r, sl]

                plsc.parallel_loop(0, VECS_PER_ROW, 1, unroll=4)(v_body)

                pltpu.async_copy(
                    obuf.at[b], out_hbm.at[pl.ds(base + c * CHUNK, CHUNK)],
                    sem_o[b])

                @pl.when(c + NBUF < n_chunks)
                def _():
                    fire_gathers(c + NBUF, b)
            return carry

        lax.fori_loop(0, n_outer, outer_body, 0, unroll=False)
        for b in range(NBUF):
            pltpu.make_async_copy(
                obuf.at[b], out_hbm.at[pl.ds(0, CHUNK)], sem_o[b]).wait()

    return gather_sum


def _ln_body(x_ref, t_ref, g_ref, b_ref, o_ref):
    e = x_ref[...] + t_ref[...]
    mu = jnp.mean(e, axis=-1, keepdims=True)
    d = e - mu
    var = jnp.mean(d * d, axis=-1, keepdims=True)
    o_ref[...] = d * lax.rsqrt(var + EPS) * g_ref[...] + b_ref[...]


def _ln_body_acc(x_ref, t_ref, g_ref, b_ref, buf_ref, o_ref):
    del buf_ref
    _ln_body(x_ref, t_ref, g_ref, b_ref, o_ref)


def _layernorm_slice(summed, type_row, gamma, beta, n_total, blk_off, buf):
    n = summed.shape[0]
    in_specs = [
        pl.BlockSpec((LN_BLK, HID), lambda i: (i, 0)),
        pl.BlockSpec((1, HID), lambda i: (0, 0)),
        pl.BlockSpec((1, HID), lambda i: (0, 0)),
        pl.BlockSpec((1, HID), lambda i: (0, 0)),
    ]
    args = [summed, type_row, gamma, beta]
    kwargs = {}
    body = _ln_body
    if buf is not None:
        in_specs.append(pl.BlockSpec(memory_space=pl.ANY))
        args.append(buf)
        kwargs["input_output_aliases"] = {4: 0}
        body = _ln_body_acc
    return pl.pallas_call(
        body,
        grid=(n // LN_BLK,),
        in_specs=in_specs,
        out_specs=pl.BlockSpec((LN_BLK, HID), lambda i: (i + blk_off, 0)),
        out_shape=jax.ShapeDtypeStruct((n_total, HID), jnp.float32),
        **kwargs,
    )(*args)


def kernel(input_ids, position_ids, word_table, pos_table, type_table,
           gamma, beta):
    b, s = input_ids.shape
    n = b * s
    ids = input_ids.reshape(n)
    pids = position_ids.reshape(n)
    h = n // NSPLIT
    gs = _make_gather_sum(h)
    type_row = type_table[0:1, :]
    g2 = gamma.reshape(1, HID)
    b2 = beta.reshape(1, HID)

    summed = [
        gs(ids[q * h:(q + 1) * h], pids[q * h:(q + 1) * h],
           word_table, pos_table)
        for q in range(NSPLIT)
    ]
    buf = None
    for q in range(NSPLIT):
        buf = _layernorm_slice(
            summed[q], type_row, g2, b2, n, q * (h // LN_BLK), buf)
    return buf.reshape(b, s, HID)
